# baseline (speedup 1.0000x reference)
import functools

import jax
import jax.numpy as jnp
from jax import lax
from jax.experimental import pallas as pl
from jax.experimental.pallas import tpu as pltpu
from jax.experimental.pallas import tpu_sc as plsc

N_NODES = 10000
N_EDGES = 160000
HEADS = 4
C = 256

NW = 32
NB = 320
NPAD = NW * NB
K = 48
EPAD = N_EDGES + 64


def _logits(h, a_ref):
    cols = []
    for hh in range(HEADS):
        hs = h[:, hh * C:(hh + 1) * C]
        cols.append(jnp.sum(hs * a_ref[hh:hh + 1, :], axis=1, keepdims=True))
    for hh in range(HEADS):
        hs = h[:, hh * C:(hh + 1) * C]
        cols.append(jnp.sum(hs * a_ref[HEADS + hh:HEADS + hh + 1, :],
                            axis=1, keepdims=True))
    return jnp.concatenate(cols, axis=1)


def _tc_in_proj(x, Wcat, brcat, A1):
    M = x.shape[0]
    bn = 1024

    def body(x_ref, w_ref, b_ref, a_ref, h_ref, xr_ref, lg_ref):
        hcat = jnp.dot(x_ref[...], w_ref[...], preferred_element_type=jnp.float32)
        h = hcat[:, :1024]
        h_ref[...] = h
        xr_ref[...] = hcat[:, 1024:] + b_ref[...]
        lg_ref[...] = _logits(h, a_ref)

    return pl.pallas_call(
        body,
        grid=(M // bn,),
        in_specs=[pl.BlockSpec((bn, 256), lambda i: (i, 0)),
                  pl.BlockSpec((256, 1536), lambda i: (0, 0)),
                  pl.BlockSpec((1, 512), lambda i: (0, 0)),
                  pl.BlockSpec((8, 256), lambda i: (0, 0))],
        out_specs=[pl.BlockSpec((bn, 1024), lambda i: (i, 0)),
                   pl.BlockSpec((bn, 512), lambda i: (i, 0)),
                   pl.BlockSpec((bn, 8), lambda i: (i, 0))],
        out_shape=[jax.ShapeDtypeStruct((M, 1024), jnp.float32),
                   jax.ShapeDtypeStruct((M, 512), jnp.float32),
                   jax.ShapeDtypeStruct((M, 8), jnp.float32)],
    )(x, Wcat, brcat, A1)


def _tc_layer_proj(aggT, xres, b, W, A):
    M = aggT.shape[1]
    bn = 1024

    def body(agg_ref, xr_ref, b_ref, w_ref, a_ref, h_ref, lg_ref):
        g = jnp.mean(agg_ref[...], axis=0) + b_ref[...] + xr_ref[...]
        g = jnp.where(g >= 0.0, g, 0.01 * g)
        h = jnp.dot(g, w_ref[...], preferred_element_type=jnp.float32)
        h_ref[...] = h
        lg_ref[...] = _logits(h, a_ref)

    return pl.pallas_call(
        body,
        grid=(M // bn,),
        in_specs=[pl.BlockSpec((HEADS, bn, 256), lambda i: (0, i, 0)),
                  pl.BlockSpec((bn, 256), lambda i: (i, 0)),
                  pl.BlockSpec((1, 256), lambda i: (0, 0)),
                  pl.BlockSpec((256, 1024), lambda i: (0, 0)),
                  pl.BlockSpec((8, 256), lambda i: (0, 0))],
        out_specs=[pl.BlockSpec((bn, 1024), lambda i: (i, 0)),
                   pl.BlockSpec((bn, 8), lambda i: (i, 0))],
        out_shape=[jax.ShapeDtypeStruct((M, 1024), jnp.float32),
                   jax.ShapeDtypeStruct((M, 8), jnp.float32)],
    )(aggT, xres, b, W, A)


def _tc_final(aggT, xres2, b):
    M = aggT.shape[1]
    bn = 1024

    def body(agg_ref, xr_ref, b_ref, o_ref):
        o_ref[...] = jnp.mean(agg_ref[...], axis=0) + b_ref[...] + xr_ref[...]

    return pl.pallas_call(
        body,
        grid=(M // bn,),
        in_specs=[pl.BlockSpec((HEADS, bn, 256), lambda i: (0, i, 0)),
                  pl.BlockSpec((bn, 256), lambda i: (i, 0)),
                  pl.BlockSpec((1, 256), lambda i: (0, 0))],
        out_specs=pl.BlockSpec((bn, 256), lambda i: (i, 0)),
        out_shape=jax.ShapeDtypeStruct((M, 256), jnp.float32),
    )(aggT, xres2, b)


def _sc_gat_edges(hflat, lgflat, src_s, dst_s, ranges):
    mesh = plsc.VectorSubcoreMesh(core_axis_name="c", subcore_axis_name="s",
                                  num_cores=2, num_subcores=16)

    @functools.partial(
        pl.kernel, mesh=mesh,
        compiler_params=pltpu.CompilerParams(needs_layout_passes=False),
        out_type=[jax.ShapeDtypeStruct((HEADS, NPAD, C), jnp.float32),
                  jax.ShapeDtypeStruct((HEADS * EPAD,), jnp.float32)],
        scratch_types=[
            pltpu.VMEM((64,), jnp.int32),
            pltpu.VMEM((K,), jnp.int32),
            pltpu.VMEM((K,), jnp.int32),
            pltpu.VMEM((K,), jnp.int32),
            pltpu.VMEM((NPAD,), jnp.float32),
            pltpu.VMEM((NPAD,), jnp.float32),
            pltpu.VMEM((K,), jnp.float32),
            pltpu.VMEM((K, C), jnp.float32),
            pltpu.VMEM((NB * HEADS,), jnp.float32),
            pltpu.VMEM((NB, C), jnp.float32),
        ],
    )
    def k(hflat_hbm, lg_hbm, src_hbm, dst_hbm, rng_hbm, out_hbm, w_hbm,
          rng_v, src_v, dst_v, gidx_v, slt_v, dlt_v, w_v, rows_v,
          den_v, acc_v):
        wid = lax.axis_index("s") * 2 + lax.axis_index("c")
        n0 = wid * NB
        pltpu.sync_copy(rng_hbm, rng_v)
        widv = jnp.full((16,), wid, jnp.int32)
        e0 = jnp.max(plsc.load_gather(rng_v, [widv]))
        e1 = jnp.max(plsc.load_gather(rng_v, [widv + 32]))
        nch = (e1 - e0 + (K - 1)) // K
        iota = jnp.arange(16, dtype=jnp.int32)
        zf = jnp.zeros((16,), jnp.float32)

        def zden(i, c):
            den_v[pl.ds(i * 16, 16)] = zf
            return c
        lax.fori_loop(0, NB * HEADS // 16, zden, 0)

        for h in range(HEADS):
            pltpu.sync_copy(lg_hbm.at[pl.ds(h * NPAD, NPAD)], slt_v)
            pltpu.sync_copy(lg_hbm.at[pl.ds((HEADS + h) * NPAD, NPAD)], dlt_v)

            def p1(ch, c):
                base = pl.multiple_of(e0 + ch * K, 8)
                pltpu.sync_copy(src_hbm.at[pl.ds(base, K)], src_v)
                pltpu.sync_copy(dst_hbm.at[pl.ds(base, K)], dst_v)
                for g in range(K // 16):
                    src16 = src_v[pl.ds(g * 16, 16)]
                    dst16 = dst_v[pl.ds(g * 16, 16)]
                    eabs = base + g * 16 + iota
                    mask = (eabs < e1) & (dst16 >= n0)
                    dls = jnp.where(mask, dst16 - n0, 0)
                    sl = plsc.load_gather(slt_v, [src16])
                    tl = plsc.load_gather(dlt_v, [dst16])
                    s = sl + tl
                    s = jnp.where(s >= 0.0, s, 0.2 * s)
                    w = jnp.exp(s)
                    plsc.addupdate_scatter(den_v, [dls * HEADS + h],
                                           jnp.where(mask, w, 0.0))
                    w_v[pl.ds(g * 16, 16)] = w
                pltpu.sync_copy(
                    w_v, w_hbm.at[pl.ds(pl.multiple_of(h * EPAD + base, 8), K)])
                return c
            lax.fori_loop(0, nch, p1, 0)

        for h in range(HEADS):
            def zacc(i, c):
                for j in range(C // 16):
                    acc_v[i, pl.ds(j * 16, 16)] = zf
                return c
            lax.fori_loop(0, NB, zacc, 0)

            def p2(ch, c):
                base = pl.multiple_of(e0 + ch * K, 8)
                pltpu.sync_copy(src_hbm.at[pl.ds(base, K)], src_v)
                pltpu.sync_copy(dst_hbm.at[pl.ds(base, K)], dst_v)
                pltpu.sync_copy(
                    w_hbm.at[pl.ds(pl.multiple_of(h * EPAD + base, 8), K)], w_v)
                for g in range(K // 16):
                    src16 = src_v[pl.ds(g * 16, 16)]
                    gidx_v[pl.ds(g * 16, 16)] = src16 * HEADS + h
                pltpu.sync_copy(hflat_hbm.at[gidx_v], rows_v)
                for g in range(K // 16):
                    dst16 = dst_v[pl.ds(g * 16, 16)]
                    eabs = base + g * 16 + iota
                    mask = (eabs < e1) & (dst16 >= n0)
                    dls = jnp.where(mask, dst16 - n0, 0)
                    w16 = w_v[pl.ds(g * 16, 16)]
                    den16 = plsc.load_gather(den_v, [dls * HEADS + h])
                    alpha = jnp.where(mask, w16 / (den16 + 1e-16), 0.0)
                    eidx = iota + g * 16

                    def fma(cc, cr):
                        for j in range(8):
                            cv = jnp.full((16,), cc * 8 + j, jnp.int32)
                            val = plsc.load_gather(rows_v, [eidx, cv])
                            plsc.addupdate_scatter(acc_v, [dls, cv],
                                                   val * alpha)
                        return cr
                    lax.fori_loop(0, C // 8, fma, 0)
                return c
            lax.fori_loop(0, nch, p2, 0)
            pltpu.sync_copy(acc_v, out_hbm.at[h, pl.ds(n0, NB)])

    return k(hflat, lgflat, src_s, dst_s, ranges)


def _a_cat(a_src, a_dst):
    return jnp.concatenate([a_src, a_dst], axis=0)


def kernel(x, edge_index, W1, a_src1, a_dst1, b1, W2, a_src2, a_dst2, b2,
           W3, a_src3, a_dst3, b3, Wr1, br1, Wr2, br2):
    f32 = jnp.float32
    src = edge_index[0].astype(jnp.int32)
    dst = edge_index[1].astype(jnp.int32)

    dst_s, src_s = lax.sort_key_val(dst, src)
    bounds = jnp.arange(NW + 1, dtype=jnp.int32) * NB
    starts = jnp.searchsorted(dst_s, bounds).astype(jnp.int32)
    ranges = jnp.concatenate([(starts[:NW] // 8) * 8, starts[1:]])
    src_s = jnp.concatenate([src_s, jnp.zeros((EPAD - N_EDGES,), jnp.int32)])
    dst_s = jnp.concatenate([dst_s, jnp.zeros((EPAD - N_EDGES,), jnp.int32)])

    xpad = jnp.pad(x, ((0, NPAD - N_NODES), (0, 0)))
    Wcat = jnp.concatenate([W1, Wr1, Wr2], axis=1)
    brcat = jnp.concatenate([br1, br2])[None, :]
    A1 = _a_cat(a_src1, a_dst1)
    A2 = _a_cat(a_src2, a_dst2)
    A3 = _a_cat(a_src3, a_dst3)

    h1, xrcat, lg1 = _tc_in_proj(xpad, Wcat, brcat, A1)
    xr1 = xrcat[:, :256]
    xr2 = xrcat[:, 256:]

    def edge_phase(hmat, lg):
        hflat = hmat.reshape(NPAD * HEADS, C)
        lgf = lg.T.reshape(8 * NPAD)
        aggT, _ = _sc_gat_edges(hflat, lgf, src_s, dst_s, ranges)
        return aggT

    aggT1 = edge_phase(h1, lg1)
    h2, lg2 = _tc_layer_proj(aggT1, xr1, b1[None, :], W2, A2)
    aggT2 = edge_phase(h2, lg2)
    h3, lg3 = _tc_layer_proj(aggT2, xr1, b2[None, :], W3, A3)
    aggT3 = edge_phase(h3, lg3)
    out = _tc_final(aggT3, xr2, b3[None, :])
    return out[:N_NODES]


# speedup vs baseline: 1.0451x; 1.0451x over previous
import functools

import jax
import jax.numpy as jnp
from jax import lax
from jax.experimental import pallas as pl
from jax.experimental.pallas import tpu as pltpu
from jax.experimental.pallas import tpu_sc as plsc

N_NODES = 10000
N_EDGES = 160000
HEADS = 4
C = 256

NW = 32
NB = 320
NPAD = NW * NB
K = 96
EPAD = N_EDGES + 96


def _logits(h, a_ref):
    cols = []
    for hh in range(HEADS):
        hs = h[:, hh * C:(hh + 1) * C]
        cols.append(jnp.sum(hs * a_ref[hh:hh + 1, :], axis=1, keepdims=True))
    for hh in range(HEADS):
        hs = h[:, hh * C:(hh + 1) * C]
        cols.append(jnp.sum(hs * a_ref[HEADS + hh:HEADS + hh + 1, :],
                            axis=1, keepdims=True))
    return jnp.concatenate(cols, axis=1)


def _tc_in_proj(x, Wcat, brcat, A1):
    M = x.shape[0]
    bn = 1024

    def body(x_ref, w_ref, b_ref, a_ref, h_ref, xr_ref, lg_ref):
        hcat = jnp.dot(x_ref[...], w_ref[...], preferred_element_type=jnp.float32)
        h = hcat[:, :1024]
        h_ref[...] = h
        xr_ref[...] = hcat[:, 1024:] + b_ref[...]
        lg_ref[...] = _logits(h, a_ref)

    return pl.pallas_call(
        body,
        grid=(M // bn,),
        in_specs=[pl.BlockSpec((bn, 256), lambda i: (i, 0)),
                  pl.BlockSpec((256, 1536), lambda i: (0, 0)),
                  pl.BlockSpec((1, 512), lambda i: (0, 0)),
                  pl.BlockSpec((8, 256), lambda i: (0, 0))],
        out_specs=[pl.BlockSpec((bn, 1024), lambda i: (i, 0)),
                   pl.BlockSpec((bn, 512), lambda i: (i, 0)),
                   pl.BlockSpec((bn, 8), lambda i: (i, 0))],
        out_shape=[jax.ShapeDtypeStruct((M, 1024), jnp.float32),
                   jax.ShapeDtypeStruct((M, 512), jnp.float32),
                   jax.ShapeDtypeStruct((M, 8), jnp.float32)],
    )(x, Wcat, brcat, A1)


def _tc_layer_proj(aggT, xres, b, W, A):
    M = aggT.shape[1]
    bn = 1024

    def body(agg_ref, xr_ref, b_ref, w_ref, a_ref, h_ref, lg_ref):
        g = jnp.mean(agg_ref[...], axis=0) + b_ref[...] + xr_ref[...]
        g = jnp.where(g >= 0.0, g, 0.01 * g)
        h = jnp.dot(g, w_ref[...], preferred_element_type=jnp.float32)
        h_ref[...] = h
        lg_ref[...] = _logits(h, a_ref)

    return pl.pallas_call(
        body,
        grid=(M // bn,),
        in_specs=[pl.BlockSpec((HEADS, bn, 256), lambda i: (0, i, 0)),
                  pl.BlockSpec((bn, 256), lambda i: (i, 0)),
                  pl.BlockSpec((1, 256), lambda i: (0, 0)),
                  pl.BlockSpec((256, 1024), lambda i: (0, 0)),
                  pl.BlockSpec((8, 256), lambda i: (0, 0))],
        out_specs=[pl.BlockSpec((bn, 1024), lambda i: (i, 0)),
                   pl.BlockSpec((bn, 8), lambda i: (i, 0))],
        out_shape=[jax.ShapeDtypeStruct((M, 1024), jnp.float32),
                   jax.ShapeDtypeStruct((M, 8), jnp.float32)],
    )(aggT, xres, b, W, A)


def _tc_final(aggT, xres2, b):
    M = aggT.shape[1]
    bn = 1024

    def body(agg_ref, xr_ref, b_ref, o_ref):
        o_ref[...] = jnp.mean(agg_ref[...], axis=0) + b_ref[...] + xr_ref[...]

    return pl.pallas_call(
        body,
        grid=(M // bn,),
        in_specs=[pl.BlockSpec((HEADS, bn, 256), lambda i: (0, i, 0)),
                  pl.BlockSpec((bn, 256), lambda i: (i, 0)),
                  pl.BlockSpec((1, 256), lambda i: (0, 0))],
        out_specs=pl.BlockSpec((bn, 256), lambda i: (i, 0)),
        out_shape=jax.ShapeDtypeStruct((M, 256), jnp.float32),
    )(aggT, xres2, b)


def _sc_gat_edges(hflat, lgflat, src_s, dst_s, ranges):
    mesh = plsc.VectorSubcoreMesh(core_axis_name="c", subcore_axis_name="s",
                                  num_cores=2, num_subcores=16)

    @functools.partial(
        pl.kernel, mesh=mesh,
        compiler_params=pltpu.CompilerParams(needs_layout_passes=False),
        out_type=[jax.ShapeDtypeStruct((HEADS, NPAD, C), jnp.float32),
                  jax.ShapeDtypeStruct((HEADS * EPAD,), jnp.float32)],
        scratch_types=[
            pltpu.VMEM((64,), jnp.int32),
            pltpu.VMEM((K,), jnp.int32),
            pltpu.VMEM((K,), jnp.int32),
            pltpu.VMEM((K,), jnp.int32),
            pltpu.VMEM((NPAD,), jnp.float32),
            pltpu.VMEM((NPAD,), jnp.float32),
            pltpu.VMEM((K,), jnp.float32),
            pltpu.VMEM((K, C), jnp.float32),
            pltpu.VMEM((NB * HEADS,), jnp.float32),
            pltpu.VMEM((NB, C), jnp.float32),
        ],
    )
    def k(hflat_hbm, lg_hbm, src_hbm, dst_hbm, rng_hbm, out_hbm, w_hbm,
          rng_v, src_v, dst_v, gidx_v, slt_v, dlt_v, w_v, rows_v,
          den_v, acc_v):
        wid = lax.axis_index("s") * 2 + lax.axis_index("c")
        n0 = wid * NB
        pltpu.sync_copy(rng_hbm, rng_v)
        widv = jnp.full((16,), wid, jnp.int32)
        e0 = jnp.max(plsc.load_gather(rng_v, [widv]))
        e1 = jnp.max(plsc.load_gather(rng_v, [widv + 32]))
        nch = (e1 - e0 + (K - 1)) // K
        iota = jnp.arange(16, dtype=jnp.int32)
        zf = jnp.zeros((16,), jnp.float32)

        def zden(i, c):
            den_v[pl.ds(i * 16, 16)] = zf
            return c
        lax.fori_loop(0, NB * HEADS // 16, zden, 0)

        for h in range(HEADS):
            pltpu.sync_copy(lg_hbm.at[pl.ds(h * NPAD, NPAD)], slt_v)
            pltpu.sync_copy(lg_hbm.at[pl.ds((HEADS + h) * NPAD, NPAD)], dlt_v)

            def p1(ch, c):
                base = pl.multiple_of(e0 + ch * K, 8)
                pltpu.sync_copy(src_hbm.at[pl.ds(base, K)], src_v)
                pltpu.sync_copy(dst_hbm.at[pl.ds(base, K)], dst_v)
                for g in range(K // 16):
                    src16 = src_v[pl.ds(g * 16, 16)]
                    dst16 = dst_v[pl.ds(g * 16, 16)]
                    eabs = base + g * 16 + iota
                    mask = (eabs < e1) & (dst16 >= n0)
                    dls = jnp.where(mask, dst16 - n0, 0)
                    sl = plsc.load_gather(slt_v, [src16])
                    tl = plsc.load_gather(dlt_v, [dst16])
                    s = sl + tl
                    s = jnp.where(s >= 0.0, s, 0.2 * s)
                    w = jnp.exp(s)
                    plsc.addupdate_scatter(den_v, [dls * HEADS + h],
                                           jnp.where(mask, w, 0.0))
                    w_v[pl.ds(g * 16, 16)] = w
                pltpu.sync_copy(
                    w_v, w_hbm.at[pl.ds(pl.multiple_of(h * EPAD + base, 8), K)])
                return c
            lax.fori_loop(0, nch, p1, 0)

        for h in range(HEADS):
            def zacc(i, c):
                for j in range(C // 16):
                    acc_v[i, pl.ds(j * 16, 16)] = zf
                return c
            lax.fori_loop(0, NB, zacc, 0)

            def p2(ch, c):
                base = pl.multiple_of(e0 + ch * K, 8)
                pltpu.sync_copy(src_hbm.at[pl.ds(base, K)], src_v)
                pltpu.sync_copy(dst_hbm.at[pl.ds(base, K)], dst_v)
                pltpu.sync_copy(
                    w_hbm.at[pl.ds(pl.multiple_of(h * EPAD + base, 8), K)], w_v)
                for g in range(K // 16):
                    src16 = src_v[pl.ds(g * 16, 16)]
                    gidx_v[pl.ds(g * 16, 16)] = src16 * HEADS + h
                pltpu.sync_copy(hflat_hbm.at[gidx_v], rows_v)
                for g in range(K // 16):
                    dst16 = dst_v[pl.ds(g * 16, 16)]
                    eabs = base + g * 16 + iota
                    mask = (eabs < e1) & (dst16 >= n0)
                    dls = jnp.where(mask, dst16 - n0, 0)
                    w16 = w_v[pl.ds(g * 16, 16)]
                    den16 = plsc.load_gather(den_v, [dls * HEADS + h])
                    alpha = jnp.where(mask, w16 / (den16 + 1e-16), 0.0)
                    eidx = iota + g * 16

                    def fma(cc, cr):
                        for j in range(8):
                            cv = jnp.full((16,), cc * 8 + j, jnp.int32)
                            val = plsc.load_gather(rows_v, [eidx, cv])
                            plsc.addupdate_scatter(acc_v, [dls, cv],
                                                   val * alpha)
                        return cr
                    lax.fori_loop(0, C // 8, fma, 0)
                return c
            lax.fori_loop(0, nch, p2, 0)
            pltpu.sync_copy(acc_v, out_hbm.at[h, pl.ds(n0, NB)])

    return k(hflat, lgflat, src_s, dst_s, ranges)


def _a_cat(a_src, a_dst):
    return jnp.concatenate([a_src, a_dst], axis=0)


def kernel(x, edge_index, W1, a_src1, a_dst1, b1, W2, a_src2, a_dst2, b2,
           W3, a_src3, a_dst3, b3, Wr1, br1, Wr2, br2):
    f32 = jnp.float32
    src = edge_index[0].astype(jnp.int32)
    dst = edge_index[1].astype(jnp.int32)

    dst_s, src_s = lax.sort_key_val(dst, src)
    bounds = jnp.arange(NW + 1, dtype=jnp.int32) * NB
    starts = jnp.searchsorted(dst_s, bounds).astype(jnp.int32)
    ranges = jnp.concatenate([(starts[:NW] // 8) * 8, starts[1:]])
    src_s = jnp.concatenate([src_s, jnp.zeros((EPAD - N_EDGES,), jnp.int32)])
    dst_s = jnp.concatenate([dst_s, jnp.zeros((EPAD - N_EDGES,), jnp.int32)])

    xpad = jnp.pad(x, ((0, NPAD - N_NODES), (0, 0)))
    Wcat = jnp.concatenate([W1, Wr1, Wr2], axis=1)
    brcat = jnp.concatenate([br1, br2])[None, :]
    A1 = _a_cat(a_src1, a_dst1)
    A2 = _a_cat(a_src2, a_dst2)
    A3 = _a_cat(a_src3, a_dst3)

    h1, xrcat, lg1 = _tc_in_proj(xpad, Wcat, brcat, A1)
    xr1 = xrcat[:, :256]
    xr2 = xrcat[:, 256:]

    def edge_phase(hmat, lg):
        hflat = hmat.reshape(NPAD * HEADS, C)
        lgf = lg.T.reshape(8 * NPAD)
        aggT, _ = _sc_gat_edges(hflat, lgf, src_s, dst_s, ranges)
        return aggT

    aggT1 = edge_phase(h1, lg1)
    h2, lg2 = _tc_layer_proj(aggT1, xr1, b1[None, :], W2, A2)
    aggT2 = edge_phase(h2, lg2)
    h3, lg3 = _tc_layer_proj(aggT2, xr1, b2[None, :], W3, A3)
    aggT3 = edge_phase(h3, lg3)
    out = _tc_final(aggT3, xr2, b3[None, :])
    return out[:N_NODES]


# speedup vs baseline: 1.1399x; 1.0907x over previous
import functools

import jax
import jax.numpy as jnp
from jax import lax
from jax.experimental import pallas as pl
from jax.experimental.pallas import tpu as pltpu
from jax.experimental.pallas import tpu_sc as plsc

N_NODES = 10000
N_EDGES = 160000
HEADS = 4
C = 256

NW = 32
NB = 320
NPAD = NW * NB
K = 96
EPAD = N_EDGES + 96


def _logits(h, a_ref):
    cols = []
    for hh in range(HEADS):
        hs = h[:, hh * C:(hh + 1) * C]
        cols.append(jnp.sum(hs * a_ref[hh:hh + 1, :], axis=1, keepdims=True))
    for hh in range(HEADS):
        hs = h[:, hh * C:(hh + 1) * C]
        cols.append(jnp.sum(hs * a_ref[HEADS + hh:HEADS + hh + 1, :],
                            axis=1, keepdims=True))
    return jnp.concatenate(cols, axis=1)


def _tc_in_proj(x, Wcat, brcat, A1):
    M = x.shape[0]
    bn = 1024

    def body(x_ref, w_ref, b_ref, a_ref, h_ref, xr_ref, lg_ref):
        hcat = jnp.dot(x_ref[...], w_ref[...], preferred_element_type=jnp.float32)
        h = hcat[:, :1024]
        h_ref[...] = h
        xr_ref[...] = hcat[:, 1024:] + b_ref[...]
        lg_ref[...] = _logits(h, a_ref)

    return pl.pallas_call(
        body,
        grid=(M // bn,),
        in_specs=[pl.BlockSpec((bn, 256), lambda i: (i, 0)),
                  pl.BlockSpec((256, 1536), lambda i: (0, 0)),
                  pl.BlockSpec((1, 512), lambda i: (0, 0)),
                  pl.BlockSpec((8, 256), lambda i: (0, 0))],
        out_specs=[pl.BlockSpec((bn, 1024), lambda i: (i, 0)),
                   pl.BlockSpec((bn, 512), lambda i: (i, 0)),
                   pl.BlockSpec((bn, 8), lambda i: (i, 0))],
        out_shape=[jax.ShapeDtypeStruct((M, 1024), jnp.float32),
                   jax.ShapeDtypeStruct((M, 512), jnp.float32),
                   jax.ShapeDtypeStruct((M, 8), jnp.float32)],
    )(x, Wcat, brcat, A1)


def _tc_layer_proj(aggT, xres, b, W, A):
    M = aggT.shape[1]
    bn = 1024

    def body(agg_ref, xr_ref, b_ref, w_ref, a_ref, h_ref, lg_ref):
        g = jnp.mean(agg_ref[...], axis=0) + b_ref[...] + xr_ref[...]
        g = jnp.where(g >= 0.0, g, 0.01 * g)
        h = jnp.dot(g, w_ref[...], preferred_element_type=jnp.float32)
        h_ref[...] = h
        lg_ref[...] = _logits(h, a_ref)

    return pl.pallas_call(
        body,
        grid=(M // bn,),
        in_specs=[pl.BlockSpec((HEADS, bn, 256), lambda i: (0, i, 0)),
                  pl.BlockSpec((bn, 256), lambda i: (i, 0)),
                  pl.BlockSpec((1, 256), lambda i: (0, 0)),
                  pl.BlockSpec((256, 1024), lambda i: (0, 0)),
                  pl.BlockSpec((8, 256), lambda i: (0, 0))],
        out_specs=[pl.BlockSpec((bn, 1024), lambda i: (i, 0)),
                   pl.BlockSpec((bn, 8), lambda i: (i, 0))],
        out_shape=[jax.ShapeDtypeStruct((M, 1024), jnp.float32),
                   jax.ShapeDtypeStruct((M, 8), jnp.float32)],
    )(aggT, xres, b, W, A)


def _tc_final(aggT, xres2, b):
    M = aggT.shape[1]
    bn = 1024

    def body(agg_ref, xr_ref, b_ref, o_ref):
        o_ref[...] = jnp.mean(agg_ref[...], axis=0) + b_ref[...] + xr_ref[...]

    return pl.pallas_call(
        body,
        grid=(M // bn,),
        in_specs=[pl.BlockSpec((HEADS, bn, 256), lambda i: (0, i, 0)),
                  pl.BlockSpec((bn, 256), lambda i: (i, 0)),
                  pl.BlockSpec((1, 256), lambda i: (0, 0))],
        out_specs=pl.BlockSpec((bn, 256), lambda i: (i, 0)),
        out_shape=jax.ShapeDtypeStruct((M, 256), jnp.float32),
    )(aggT, xres2, b)


def _sc_gat_edges(hflat, lgflat, src_s, dst_s, ranges):
    mesh = plsc.VectorSubcoreMesh(core_axis_name="c", subcore_axis_name="s",
                                  num_cores=2, num_subcores=16)

    @functools.partial(
        pl.kernel, mesh=mesh,
        compiler_params=pltpu.CompilerParams(needs_layout_passes=False),
        out_type=[jax.ShapeDtypeStruct((HEADS, NPAD, C), jnp.float32),
                  jax.ShapeDtypeStruct((HEADS * EPAD,), jnp.float32)],
        scratch_types=[
            pltpu.VMEM((64,), jnp.int32),
            pltpu.VMEM((K,), jnp.int32),
            pltpu.VMEM((K,), jnp.int32),
            pltpu.VMEM((K,), jnp.int32),
            pltpu.VMEM((NPAD,), jnp.float32),
            pltpu.VMEM((NPAD,), jnp.float32),
            pltpu.VMEM((K,), jnp.float32),
            pltpu.VMEM((K, C), jnp.float32),
            pltpu.VMEM((NB * HEADS,), jnp.float32),
            pltpu.VMEM((NB, C), jnp.float32),
        ],
    )
    def k(hflat_hbm, lg_hbm, src_hbm, dst_hbm, rng_hbm, out_hbm, w_hbm,
          rng_v, src_v, dst_v, gidx_v, slt_v, dlt_v, w_v, rows_v,
          den_v, acc_v):
        wid = lax.axis_index("s") * 2 + lax.axis_index("c")
        n0 = wid * NB
        pltpu.sync_copy(rng_hbm, rng_v)
        widv = jnp.full((16,), wid, jnp.int32)
        e0 = jnp.max(plsc.load_gather(rng_v, [widv]))
        e1 = jnp.max(plsc.load_gather(rng_v, [widv + 32]))
        nch = (e1 - e0 + (K - 1)) // K
        iota = jnp.arange(16, dtype=jnp.int32)
        zf = jnp.zeros((16,), jnp.float32)

        def zden(i, c):
            den_v[pl.ds(i * 16, 16)] = zf
            return c
        lax.fori_loop(0, NB * HEADS // 16, zden, 0)

        for h in range(HEADS):
            pltpu.sync_copy(lg_hbm.at[pl.ds(h * NPAD, NPAD)], slt_v)
            pltpu.sync_copy(lg_hbm.at[pl.ds((HEADS + h) * NPAD, NPAD)], dlt_v)

            def p1(ch, c):
                base = pl.multiple_of(e0 + ch * K, 8)
                pltpu.sync_copy(src_hbm.at[pl.ds(base, K)], src_v)
                pltpu.sync_copy(dst_hbm.at[pl.ds(base, K)], dst_v)
                for g in range(K // 16):
                    src16 = src_v[pl.ds(g * 16, 16)]
                    dst16 = dst_v[pl.ds(g * 16, 16)]
                    eabs = base + g * 16 + iota
                    mask = (eabs < e1) & (dst16 >= n0)
                    dls = jnp.where(mask, dst16 - n0, 0)
                    sl = plsc.load_gather(slt_v, [src16])
                    tl = plsc.load_gather(dlt_v, [dst16])
                    s = sl + tl
                    s = jnp.where(s >= 0.0, s, 0.2 * s)
                    w = jnp.exp(s)
                    plsc.addupdate_scatter(den_v, [dls * HEADS + h],
                                           jnp.where(mask, w, 0.0))
                    w_v[pl.ds(g * 16, 16)] = w
                pltpu.sync_copy(
                    w_v, w_hbm.at[pl.ds(pl.multiple_of(h * EPAD + base, 8), K)])
                return c
            lax.fori_loop(0, nch, p1, 0)

        for h in range(HEADS):
            def zacc(i, c):
                for j in range(C // 16):
                    acc_v[i, pl.ds(j * 16, 16)] = zf
                return c
            lax.fori_loop(0, NB, zacc, 0)

            def p2(ch, c):
                base = pl.multiple_of(e0 + ch * K, 8)
                pltpu.sync_copy(src_hbm.at[pl.ds(base, K)], src_v)
                pltpu.sync_copy(dst_hbm.at[pl.ds(base, K)], dst_v)
                pltpu.sync_copy(
                    w_hbm.at[pl.ds(pl.multiple_of(h * EPAD + base, 8), K)], w_v)
                for g in range(K // 16):
                    src16 = src_v[pl.ds(g * 16, 16)]
                    gidx_v[pl.ds(g * 16, 16)] = src16 * HEADS + h
                pltpu.sync_copy(hflat_hbm.at[gidx_v], rows_v)
                for g in range(K // 16):
                    eidx = iota * (K // 16) + g
                    dst16 = plsc.load_gather(dst_v, [eidx])
                    eabs = base + eidx
                    mask = (eabs < e1) & (dst16 >= n0)
                    dls = jnp.where(mask, dst16 - n0, 0)
                    w16 = plsc.load_gather(w_v, [eidx])
                    den16 = plsc.load_gather(den_v, [dls * HEADS + h])
                    alpha = jnp.where(mask, w16 / (den16 + 1e-16), 0.0)

                    def fma(cc, cr):
                        for j in range(8):
                            cv = jnp.full((16,), cc * 8 + j, jnp.int32)
                            val = plsc.load_gather(rows_v, [eidx, cv])
                            plsc.addupdate_scatter(acc_v, [dls, cv],
                                                   val * alpha)
                        return cr
                    lax.fori_loop(0, C // 8, fma, 0)
                return c
            lax.fori_loop(0, nch, p2, 0)
            pltpu.sync_copy(acc_v, out_hbm.at[h, pl.ds(n0, NB)])

    return k(hflat, lgflat, src_s, dst_s, ranges)


def _a_cat(a_src, a_dst):
    return jnp.concatenate([a_src, a_dst], axis=0)


def kernel(x, edge_index, W1, a_src1, a_dst1, b1, W2, a_src2, a_dst2, b2,
           W3, a_src3, a_dst3, b3, Wr1, br1, Wr2, br2):
    f32 = jnp.float32
    src = edge_index[0].astype(jnp.int32)
    dst = edge_index[1].astype(jnp.int32)

    dst_s, src_s = lax.sort_key_val(dst, src)
    bounds = jnp.arange(NW + 1, dtype=jnp.int32) * NB
    starts = jnp.searchsorted(dst_s, bounds).astype(jnp.int32)
    ranges = jnp.concatenate([(starts[:NW] // 8) * 8, starts[1:]])
    src_s = jnp.concatenate([src_s, jnp.zeros((EPAD - N_EDGES,), jnp.int32)])
    dst_s = jnp.concatenate([dst_s, jnp.zeros((EPAD - N_EDGES,), jnp.int32)])

    xpad = jnp.pad(x, ((0, NPAD - N_NODES), (0, 0)))
    Wcat = jnp.concatenate([W1, Wr1, Wr2], axis=1)
    brcat = jnp.concatenate([br1, br2])[None, :]
    A1 = _a_cat(a_src1, a_dst1)
    A2 = _a_cat(a_src2, a_dst2)
    A3 = _a_cat(a_src3, a_dst3)

    h1, xrcat, lg1 = _tc_in_proj(xpad, Wcat, brcat, A1)
    xr1 = xrcat[:, :256]
    xr2 = xrcat[:, 256:]

    def edge_phase(hmat, lg):
        hflat = hmat.reshape(NPAD * HEADS, C)
        lgf = lg.T.reshape(8 * NPAD)
        aggT, _ = _sc_gat_edges(hflat, lgf, src_s, dst_s, ranges)
        return aggT

    aggT1 = edge_phase(h1, lg1)
    h2, lg2 = _tc_layer_proj(aggT1, xr1, b1[None, :], W2, A2)
    aggT2 = edge_phase(h2, lg2)
    h3, lg3 = _tc_layer_proj(aggT2, xr1, b2[None, :], W3, A3)
    aggT3 = edge_phase(h3, lg3)
    out = _tc_final(aggT3, xr2, b3[None, :])
    return out[:N_NODES]
